# Initial kernel scaffold; baseline (speedup 1.0000x reference)
import jax
import jax.numpy as jnp
from jax.experimental import pallas as pl


def kernel(x, edge_index, W1, b1, W2, b2, Wlin, blin):
    raise NotImplementedError("write your pallas kernel here")


# speedup vs baseline: 4.8991x; 4.8991x over previous
import functools

import jax
import jax.numpy as jnp
from jax import lax
from jax.experimental import pallas as pl
from jax.experimental.pallas import tpu as pltpu
from jax.experimental.pallas import tpu_sc as plsc

N = 10000
D = 128
NC, NS, L = 2, 16, 16
NW = NC * NS
NPR = 640
HPR = NPR // 2
NPAIR = NW // 2
NPAD = NPAIR * NPR

CHUNKP = 6400
PCK_CH = 1024
GU = 128
PADF = PCK_CH + GU

ROW_BITS = 14
ROW_MASK = (1 << ROW_BITS) - 1


def _mesh():
    return plsc.VectorSubcoreMesh(core_axis_name="c", subcore_axis_name="s")


def _ids():
    c = lax.axis_index("c")
    s = lax.axis_index("s")
    w = c * NS + s
    pid = c * (NS // 2) + lax.shift_right_logical(s, 1)
    half = s & 1
    return w, pid, half


def _sc_pre_body(col_hbm, row_hbm, pck_hbm, cnt_hbm, deg_hbm, degx_hbm,
                 colv0, rowv0, colv1, rowv1, stag, hist, histp, cntv,
                 sem0, sem1):
    w, pid, half = _ids()
    base = pid * NPR
    base_al = pl.multiple_of(base, 8)
    cap = pck_hbm.shape[0] // NW
    woff = pl.multiple_of(w * cap, 8)
    e_half = (col_hbm.shape[0] - CHUNKP) // 2
    eoff = pl.multiple_of(half * e_half, 8)
    nchunks = e_half // CHUNKP

    zeros16 = jnp.zeros((L,), jnp.float32)
    for j in range(NPR // L):
        hist[pl.ds(j * L, L)] = zeros16

    def start_load(c, cv, rv, sem):
        off = eoff + c * CHUNKP
        pltpu.async_copy(col_hbm.at[pl.ds(off, CHUNKP)], cv, sem)
        pltpu.async_copy(row_hbm.at[pl.ds(off, CHUNKP)], rv, sem)

    def wait_load(c, cv, rv, sem):
        off = eoff + c * CHUNKP
        pltpu.make_async_copy(col_hbm.at[pl.ds(off, CHUNKP)], cv, sem).wait()
        pltpu.make_async_copy(row_hbm.at[pl.ds(off, CHUNKP)], rv, sem).wait()

    ones16 = jnp.ones((L,), jnp.float32)

    def scan_chunk(cv, rv, total):
        def group_body(g, k):
            for h in range(2):
                sl = pl.ds((2 * g + h) * L, L)
                cols = cv[sl]
                rows = rv[sl]
                lcl = cols - base
                m = plsc.bitcast(lcl, jnp.uint32) < jnp.uint32(NPR)
                packed = rows | (lcl << ROW_BITS)
                cs = plsc.cumsum(m.astype(jnp.int32))
                pos = k + cs - 1
                plsc.store_scatter(stag, [pos], packed, mask=m)
                plsc.addupdate_scatter(hist, [lcl], ones16, mask=m)
                k = k + plsc.all_reduce_population_count(m)[0]
            return k

        k = lax.fori_loop(0, CHUNKP // (2 * L), group_body, jnp.int32(0))
        stag[pl.ds(k, L)] = jnp.full((L,), base, jnp.int32)
        k8 = ((k + 7) // 8) * 8
        dst = pck_hbm.at[pl.ds(woff + pl.multiple_of(total, 8), CHUNKP)]
        pltpu.sync_copy(stag.at[pl.ds(0, CHUNKP)], dst)
        return total + k8

    start_load(jnp.int32(0), colv0, rowv0, sem0)

    def pair_body(kp, total):
        c = 2 * kp
        start_load(c + 1, colv1, rowv1, sem1)
        wait_load(c, colv0, rowv0, sem0)
        total = scan_chunk(colv0, rowv0, total)
        start_load(c + 2, colv0, rowv0, sem0)
        wait_load(c + 1, colv1, rowv1, sem1)
        return scan_chunk(colv1, rowv1, total)

    total = lax.fori_loop(0, nchunks // 2, pair_body, jnp.int32(0))
    wait_load(jnp.int32(nchunks), colv0, rowv0, sem0)

    padv = jnp.full((L,), base, jnp.int32)
    for t in range(PADF // L):
        stag[pl.ds(t * L, L)] = padv
    pltpu.sync_copy(stag.at[pl.ds(0, PADF)],
                    pck_hbm.at[pl.ds(woff + pl.multiple_of(total, 8), PADF)])

    cntv[...] = jnp.full((L,), total, jnp.int32)
    pltpu.sync_copy(cntv, cnt_hbm.at[pl.ds(pl.multiple_of(w * L, 8), L)])

    pltpu.sync_copy(hist, degx_hbm.at[pl.ds(pl.multiple_of(w * NPR, 8), NPR)])
    plsc.subcore_barrier()
    pltpu.sync_copy(degx_hbm.at[pl.ds(pl.multiple_of((w ^ 1) * NPR, 8), NPR)],
                    histp)
    for j in range(NPR // L):
        sl = pl.ds(j * L, L)
        hist[sl] = hist[sl] + histp[sl] + 1.0
    pltpu.sync_copy(hist, deg_hbm.at[pl.ds(base_al, NPR)])


def _make_sc_pre(cap):
    return functools.partial(
        pl.kernel,
        compiler_params=pltpu.CompilerParams(needs_layout_passes=False),
        out_type=(
            jax.ShapeDtypeStruct((NW * cap,), jnp.int32),
            jax.ShapeDtypeStruct((NW * L,), jnp.int32),
            jax.ShapeDtypeStruct((NPAD,), jnp.float32),
            jax.ShapeDtypeStruct((NW * NPR,), jnp.float32),
        ),
        mesh=_mesh(),
        scratch_types=[
            pltpu.VMEM((CHUNKP,), jnp.int32),
            pltpu.VMEM((CHUNKP,), jnp.int32),
            pltpu.VMEM((CHUNKP,), jnp.int32),
            pltpu.VMEM((CHUNKP,), jnp.int32),
            pltpu.VMEM((CHUNKP + 2 * L,), jnp.int32),
            pltpu.VMEM((NPR,), jnp.float32),
            pltpu.VMEM((NPR,), jnp.float32),
            pltpu.VMEM((L,), jnp.int32),
            pltpu.SemaphoreType.DMA,
            pltpu.SemaphoreType.DMA,
        ],
    )(_sc_pre_body)


def _sc_max_body(g_hbm, pck_hbm, cnt_hbm, m_hbm, xch_hbm,
                 accv, pckv, ridx0, ridx1, lclv0, lclv1, rows0, rows1,
                 cntv, sem0, sem1):
    w, pid, half = _ids()
    base = pid * NPR
    base_al = pl.multiple_of(base, 8)
    cap = pck_hbm.shape[0] // NW
    woff = pl.multiple_of(w * cap, 8)
    pltpu.sync_copy(g_hbm.at[pl.ds(base_al, NPR)], accv)
    pltpu.sync_copy(cnt_hbm.at[pl.ds(pl.multiple_of(w * L, 8), L)], cntv)
    cnt = cntv[...][0]
    nch = (cnt + (PCK_CH - 1)) // PCK_CH
    nu = nch * (PCK_CH // GU)

    def load_chunk(u):
        @pl.when((u & 7) == 0)
        def _():
            cidx = jnp.clip(lax.shift_right_logical(u, 3), 0, nch - 1)
            pltpu.sync_copy(
                pck_hbm.at[pl.ds(woff + cidx * PCK_CH, PCK_CH)], pckv)

    def unpack(u, rx, lx):
        sub = u & 7
        for gg in range(GU // L):
            pv = pckv[pl.ds(sub * GU + gg * L, L)]
            rx[pl.ds(gg * L, L)] = pv & ROW_MASK
            lx[pl.ds(gg * L, L)] = lax.shift_right_logical(pv, ROW_BITS)

    def accum(lx, rb):
        def edge_body(j, lc):
            nxt = lx[pl.ds(j + 1, L)][0]
            slices = [pl.ds(dv * L, L) for dv in range(D // L)]
            rs = [rb[j, sl] for sl in slices]
            accs = [accv[lc, sl] for sl in slices]
            for sl, a, r in zip(slices, accs, rs):
                accv[lc, sl] = jnp.maximum(a, r)
            return nxt

        lc0 = lx[pl.ds(0, L)][0]
        lax.fori_loop(0, GU, edge_body, lc0)

    load_chunk(jnp.int32(0))
    unpack(jnp.int32(0), ridx0, lclv0)
    pltpu.async_copy(g_hbm.at[ridx0], rows0, sem0)

    def pair_body(k, _):
        u = 2 * k
        unpack(u + 1, ridx1, lclv1)
        pltpu.async_copy(g_hbm.at[ridx1], rows1, sem1)
        pltpu.make_async_copy(g_hbm.at[ridx0], rows0, sem0).wait()
        accum(lclv0, rows0)
        load_chunk(u + 2)
        unpack(u + 2, ridx0, lclv0)
        pltpu.async_copy(g_hbm.at[ridx0], rows0, sem0)
        pltpu.make_async_copy(g_hbm.at[ridx1], rows1, sem1).wait()
        accum(lclv1, rows1)
        return 0

    lax.fori_loop(0, lax.shift_right_logical(nu, 1), pair_body, 0)
    pltpu.make_async_copy(g_hbm.at[ridx0], rows0, sem0).wait()

    oth = pl.multiple_of((1 - half) * HPR, 8)
    my = pl.multiple_of(half * HPR, 8)
    xw = pl.multiple_of(w * HPR, 8)
    xp = pl.multiple_of((w ^ 1) * HPR, 8)
    pltpu.sync_copy(accv.at[pl.ds(oth, HPR)], xch_hbm.at[pl.ds(xw, HPR)])
    plsc.subcore_barrier()
    MC = 80
    for cc in range(HPR // MC):
        pltpu.sync_copy(xch_hbm.at[pl.ds(xp + cc * MC, MC)],
                        rows0.at[pl.ds(0, MC)])

        def merge_body(j, _):
            r = my + cc * MC + j
            slices = [pl.ds(dv * L, L) for dv in range(D // L)]
            rs = [rows0[j, sl] for sl in slices]
            accs = [accv[r, sl] for sl in slices]
            for sl, a, x in zip(slices, accs, rs):
                accv[r, sl] = jnp.maximum(a, x)
            return 0

        lax.fori_loop(0, MC, merge_body, 0)
    pltpu.sync_copy(accv.at[pl.ds(my, HPR)],
                    m_hbm.at[pl.ds(pl.multiple_of(base + half * HPR, 8),
                                   HPR)])


def _make_sc_max(cap):
    return functools.partial(
        pl.kernel,
        compiler_params=pltpu.CompilerParams(needs_layout_passes=False),
        out_type=(
            jax.ShapeDtypeStruct((NPAD, D), jnp.float32),
            jax.ShapeDtypeStruct((NW * HPR, D), jnp.float32),
        ),
        mesh=_mesh(),
        scratch_types=[
            pltpu.VMEM((NPR, D), jnp.float32),
            pltpu.VMEM((PCK_CH,), jnp.int32),
            pltpu.VMEM((GU,), jnp.int32),
            pltpu.VMEM((GU,), jnp.int32),
            pltpu.VMEM((GU + L,), jnp.int32),
            pltpu.VMEM((GU + L,), jnp.int32),
            pltpu.VMEM((GU, D), jnp.float32),
            pltpu.VMEM((GU, D), jnp.float32),
            pltpu.VMEM((L,), jnp.int32),
            pltpu.SemaphoreType.DMA,
            pltpu.SemaphoreType.DMA,
        ],
    )(_sc_max_body)


_BR = 1280


def _tc1_body(x_ref, w_ref, deg_ref, o_ref):
    dis = lax.rsqrt(deg_ref[...])
    o_ref[...] = dis * jnp.dot(x_ref[...], w_ref[...],
                               preferred_element_type=jnp.float32)


def _tc2_body(m_ref, deg_ref, b_ref, w_ref, o_ref):
    dis = lax.rsqrt(deg_ref[...])
    h = jnp.maximum(dis * m_ref[...] + b_ref[...], 0.0)
    o_ref[...] = dis * jnp.dot(h, w_ref[...],
                               preferred_element_type=jnp.float32)


def _tc3_body(m_ref, deg_ref, b_ref, w_ref, blin_ref, o_ref, *, n_classes):
    dis = lax.rsqrt(deg_ref[...])
    h = jnp.maximum(dis * m_ref[...] + b_ref[...], 0.0)
    logits = jnp.dot(h, w_ref[...], preferred_element_type=jnp.float32)
    logits = logits + blin_ref[...]
    colid = lax.broadcasted_iota(jnp.int32, logits.shape, 1)
    valid = colid < n_classes
    neg = jnp.float32(-3.0e38)
    logits = jnp.where(valid, logits, neg)
    mx = jnp.max(logits, axis=-1, keepdims=True)
    e = jnp.where(valid, jnp.exp(logits - mx), 0.0)
    s = jnp.sum(e, axis=-1, keepdims=True)
    o_ref[...] = logits - mx - jnp.log(s)


def _tc_call(body, nrows, ins, row_blocked, out_cols=D):
    grid = (nrows // _BR,)
    in_specs = []
    for a, blocked in zip(ins, row_blocked):
        if blocked:
            in_specs.append(pl.BlockSpec((_BR, a.shape[1]), lambda i: (i, 0)))
        else:
            in_specs.append(pl.BlockSpec(a.shape, lambda i: (0, 0)))
    return pl.pallas_call(
        body,
        grid=grid,
        in_specs=in_specs,
        out_specs=pl.BlockSpec((_BR, out_cols), lambda i: (i, 0)),
        out_shape=jax.ShapeDtypeStruct((nrows, out_cols), jnp.float32),
    )(*ins)


def kernel(x, edge_index, W1, b1, W2, b2, Wlin, blin):
    n, d = x.shape
    e = edge_index.shape[1]
    n_classes = Wlin.shape[1]
    assert n == N and d == D

    e_pad = ((e + 4 * CHUNKP - 1) // (4 * CHUNKP)) * (4 * CHUNKP)
    e_half = e_pad // 2
    cap = e_half + 8 * (e_half // CHUNKP) + CHUNKP + PADF
    cap = ((cap + 7) // 8) * 8

    row = edge_index[0].astype(jnp.int32)
    col = edge_index[1].astype(jnp.int32)
    row = jnp.concatenate(
        [row, jnp.zeros((e_pad + CHUNKP - e,), jnp.int32)])
    col = jnp.concatenate(
        [col, jnp.full((e_pad - e,), NPAD - 1, jnp.int32),
         jnp.zeros((CHUNKP,), jnp.int32)])

    xp = jnp.pad(x, ((0, NPAD - n), (0, 0)))

    pck, cnts, deg, _ = _make_sc_pre(cap)(col, row)
    degc = deg.reshape(NPAD, 1)

    b1r = b1.reshape(1, D)
    b2r = b2.reshape(1, D)
    wl = jnp.pad(Wlin, ((0, 0), (0, D - n_classes)))
    blr = jnp.pad(blin, (0, D - n_classes)).reshape(1, D)

    sc_max = _make_sc_max(cap)

    g1 = _tc_call(_tc1_body, NPAD, (xp, W1, degc), (True, False, True))
    m1, _ = sc_max(g1, pck, cnts)
    g2 = _tc_call(_tc2_body, NPAD, (m1, degc, b1r, W2),
                  (True, True, False, False))
    m2, _ = sc_max(g2, pck, cnts)
    out = _tc_call(functools.partial(_tc3_body, n_classes=n_classes),
                   NPAD, (m2, degc, b2r, wl, blr),
                   (True, True, False, False, False))
    return out[:n, :n_classes]


# speedup vs baseline: 15.9884x; 3.2635x over previous
import functools

import jax
import jax.numpy as jnp
from jax import lax
from jax.experimental import pallas as pl
from jax.experimental.pallas import tpu as pltpu
from jax.experimental.pallas import tpu_sc as plsc

N = 10000
D = 128
NC, NS, L = 2, 16, 16
NW = NC * NS
NPR = 640
HPR = NPR // 2
NPAIR = NW // 2
NPAD = NPAIR * NPR

CHUNKP = 6400
PCK_CH = 2048
GU = 128
UPC = PCK_CH // GU
UPC_SHIFT = UPC.bit_length() - 1
PADF = PCK_CH + GU

ROW_BITS = 14
ROW_MASK = (1 << ROW_BITS) - 1
SMALLF = 512


def _mesh():
    return plsc.VectorSubcoreMesh(core_axis_name="c", subcore_axis_name="s")


def _ids():
    c = lax.axis_index("c")
    s = lax.axis_index("s")
    w = c * NS + s
    pid = c * (NS // 2) + lax.shift_right_logical(s, 1)
    half = s & 1
    return w, pid, half


def _sc_pre_body(col_hbm, row_hbm, pck_hbm, cnt_hbm, deg_hbm, degx_hbm,
                 colv0, rowv0, colv1, rowv1, stag, hist, histp, cntv,
                 sem0, sem1):
    w, pid, half = _ids()
    base = pid * NPR
    base_al = pl.multiple_of(base, 8)
    cap = pck_hbm.shape[0] // NW
    woff = pl.multiple_of(w * cap, 8)
    e_half = (col_hbm.shape[0] - CHUNKP) // 2
    eoff = pl.multiple_of(half * e_half, 8)
    nchunks = e_half // CHUNKP

    zeros16 = jnp.zeros((L,), jnp.float32)
    for j in range(NPR // L):
        hist[pl.ds(j * L, L)] = zeros16

    def start_load(c, cv, rv, sem):
        off = eoff + c * CHUNKP
        pltpu.async_copy(col_hbm.at[pl.ds(off, CHUNKP)], cv, sem)
        pltpu.async_copy(row_hbm.at[pl.ds(off, CHUNKP)], rv, sem)

    def wait_load(c, cv, rv, sem):
        off = eoff + c * CHUNKP
        pltpu.make_async_copy(col_hbm.at[pl.ds(off, CHUNKP)], cv, sem).wait()
        pltpu.make_async_copy(row_hbm.at[pl.ds(off, CHUNKP)], rv, sem).wait()

    ones16 = jnp.ones((L,), jnp.float32)

    def scan_chunk(cv, rv, total):
        def group_body(g, k):
            for h in range(2):
                sl = pl.ds((2 * g + h) * L, L)
                cols = cv[sl]
                rows = rv[sl]
                lcl = cols - base
                m = plsc.bitcast(lcl, jnp.uint32) < jnp.uint32(NPR)
                packed = rows | (lcl << ROW_BITS)
                cs = plsc.cumsum(m.astype(jnp.int32))
                pos = k + cs - 1
                plsc.store_scatter(stag, [pos], packed, mask=m)
                plsc.addupdate_scatter(hist, [lcl], ones16, mask=m)
                k = k + plsc.all_reduce_population_count(m)[0]
            return k

        k = lax.fori_loop(0, CHUNKP // (2 * L), group_body, jnp.int32(0))
        stag[pl.ds(k, L)] = jnp.full((L,), base, jnp.int32)
        k8 = ((k + 7) // 8) * 8
        off = pl.multiple_of(total, 8)

        @pl.when(k8 <= SMALLF)
        def _():
            pltpu.sync_copy(stag.at[pl.ds(0, SMALLF)],
                            pck_hbm.at[pl.ds(woff + off, SMALLF)])

        @pl.when(k8 > SMALLF)
        def _():
            pltpu.sync_copy(stag.at[pl.ds(0, CHUNKP)],
                            pck_hbm.at[pl.ds(woff + off, CHUNKP)])

        return total + k8

    start_load(jnp.int32(0), colv0, rowv0, sem0)

    def pair_body(kp, total):
        c = 2 * kp
        start_load(c + 1, colv1, rowv1, sem1)
        wait_load(c, colv0, rowv0, sem0)
        total = scan_chunk(colv0, rowv0, total)
        start_load(c + 2, colv0, rowv0, sem0)
        wait_load(c + 1, colv1, rowv1, sem1)
        return scan_chunk(colv1, rowv1, total)

    total = lax.fori_loop(0, nchunks // 2, pair_body, jnp.int32(0))
    wait_load(jnp.int32(nchunks), colv0, rowv0, sem0)

    padv = jnp.full((L,), base, jnp.int32)
    for t in range(PADF // L):
        stag[pl.ds(t * L, L)] = padv
    pltpu.sync_copy(stag.at[pl.ds(0, PADF)],
                    pck_hbm.at[pl.ds(woff + pl.multiple_of(total, 8), PADF)])

    cntv[...] = jnp.full((L,), total, jnp.int32)
    pltpu.sync_copy(cntv, cnt_hbm.at[pl.ds(pl.multiple_of(w * L, 8), L)])

    pltpu.sync_copy(hist, degx_hbm.at[pl.ds(pl.multiple_of(w * NPR, 8), NPR)])
    plsc.subcore_barrier()
    pltpu.sync_copy(degx_hbm.at[pl.ds(pl.multiple_of((w ^ 1) * NPR, 8), NPR)],
                    histp)
    for j in range(NPR // L):
        sl = pl.ds(j * L, L)
        hist[sl] = hist[sl] + histp[sl] + 1.0
    pltpu.sync_copy(hist, deg_hbm.at[pl.ds(base_al, NPR)])


def _make_sc_pre(cap):
    return functools.partial(
        pl.kernel,
        compiler_params=pltpu.CompilerParams(needs_layout_passes=False),
        out_type=(
            jax.ShapeDtypeStruct((NW * cap,), jnp.int32),
            jax.ShapeDtypeStruct((NW * L,), jnp.int32),
            jax.ShapeDtypeStruct((NPAD,), jnp.float32),
            jax.ShapeDtypeStruct((NW * NPR,), jnp.float32),
        ),
        mesh=_mesh(),
        scratch_types=[
            pltpu.VMEM((CHUNKP,), jnp.int32),
            pltpu.VMEM((CHUNKP,), jnp.int32),
            pltpu.VMEM((CHUNKP,), jnp.int32),
            pltpu.VMEM((CHUNKP,), jnp.int32),
            pltpu.VMEM((CHUNKP + 2 * L,), jnp.int32),
            pltpu.VMEM((NPR,), jnp.float32),
            pltpu.VMEM((NPR,), jnp.float32),
            pltpu.VMEM((L,), jnp.int32),
            pltpu.SemaphoreType.DMA,
            pltpu.SemaphoreType.DMA,
        ],
    )(_sc_pre_body)


def _sc_max_body(g_hbm, pck_hbm, cnt_hbm, m_hbm, xch_hbm,
                 accv, pckv, ridx0, ridx1, lclv0, lclv1, rows0, rows1,
                 cntv, sem0, sem1):
    w, pid, half = _ids()
    base = pid * NPR
    base_al = pl.multiple_of(base, 8)
    cap = pck_hbm.shape[0] // NW
    woff = pl.multiple_of(w * cap, 8)
    pltpu.sync_copy(g_hbm.at[pl.ds(base_al, NPR)], accv)
    pltpu.sync_copy(cnt_hbm.at[pl.ds(pl.multiple_of(w * L, 8), L)], cntv)
    cnt = cntv[...][0]
    nu = (((cnt + GU - 1) // GU + 1) // 2) * 2
    nch = (nu + UPC - 1) // UPC

    def load_chunk(u):
        @pl.when((u & (UPC - 1)) == 0)
        def _():
            cidx = jnp.maximum(
                jnp.minimum(lax.shift_right_logical(u, UPC_SHIFT), nch - 1),
                0)
            pltpu.sync_copy(
                pck_hbm.at[pl.ds(woff + cidx * PCK_CH, PCK_CH)], pckv)

    def unpack(u, rx, lx):
        sub = u & (UPC - 1)
        for gg in range(GU // L):
            pv = pckv[pl.ds(sub * GU + gg * L, L)]
            rx[pl.ds(gg * L, L)] = pv & ROW_MASK
            lx[pl.ds(gg * L, L)] = lax.shift_right_logical(pv, ROW_BITS)

    def accum(lx, rb):
        def edge_body(j, lc):
            nxt = lx[pl.ds(j + 1, L)][0]
            slices = [pl.ds(dv * L, L) for dv in range(D // L)]
            rs = [rb[j, sl] for sl in slices]
            accs = [accv[lc, sl] for sl in slices]
            for sl, a, r in zip(slices, accs, rs):
                accv[lc, sl] = jnp.maximum(a, r)
            return nxt

        lc0 = lx[pl.ds(0, L)][0]
        lax.fori_loop(0, GU, edge_body, lc0)

    load_chunk(jnp.int32(0))
    unpack(jnp.int32(0), ridx0, lclv0)
    pltpu.async_copy(g_hbm.at[ridx0], rows0, sem0)

    def pair_body(k, _):
        u = 2 * k
        unpack(u + 1, ridx1, lclv1)
        pltpu.async_copy(g_hbm.at[ridx1], rows1, sem1)
        pltpu.make_async_copy(g_hbm.at[ridx0], rows0, sem0).wait()
        accum(lclv0, rows0)
        load_chunk(u + 2)
        unpack(u + 2, ridx0, lclv0)
        pltpu.async_copy(g_hbm.at[ridx0], rows0, sem0)
        pltpu.make_async_copy(g_hbm.at[ridx1], rows1, sem1).wait()
        accum(lclv1, rows1)
        return 0

    lax.fori_loop(0, lax.shift_right_logical(nu, 1), pair_body, 0)
    pltpu.make_async_copy(g_hbm.at[ridx0], rows0, sem0).wait()

    oth = pl.multiple_of((1 - half) * HPR, 8)
    my = pl.multiple_of(half * HPR, 8)
    xw = pl.multiple_of(w * HPR, 8)
    xp = pl.multiple_of((w ^ 1) * HPR, 8)
    pltpu.sync_copy(accv.at[pl.ds(oth, HPR)], xch_hbm.at[pl.ds(xw, HPR)])
    plsc.subcore_barrier()
    MC = 80
    for cc in range(HPR // MC):
        pltpu.sync_copy(xch_hbm.at[pl.ds(xp + cc * MC, MC)],
                        rows0.at[pl.ds(0, MC)])

        def merge_body(j, _):
            r = my + cc * MC + j
            slices = [pl.ds(dv * L, L) for dv in range(D // L)]
            rs = [rows0[j, sl] for sl in slices]
            accs = [accv[r, sl] for sl in slices]
            for sl, a, x in zip(slices, accs, rs):
                accv[r, sl] = jnp.maximum(a, x)
            return 0

        lax.fori_loop(0, MC, merge_body, 0)
    pltpu.sync_copy(accv.at[pl.ds(my, HPR)],
                    m_hbm.at[pl.ds(pl.multiple_of(base + half * HPR, 8),
                                   HPR)])


def _make_sc_max(cap):
    return functools.partial(
        pl.kernel,
        compiler_params=pltpu.CompilerParams(needs_layout_passes=False),
        out_type=(
            jax.ShapeDtypeStruct((NPAD, D), jnp.float32),
            jax.ShapeDtypeStruct((NW * HPR, D), jnp.float32),
        ),
        mesh=_mesh(),
        scratch_types=[
            pltpu.VMEM((NPR, D), jnp.float32),
            pltpu.VMEM((PCK_CH,), jnp.int32),
            pltpu.VMEM((GU,), jnp.int32),
            pltpu.VMEM((GU,), jnp.int32),
            pltpu.VMEM((GU + L,), jnp.int32),
            pltpu.VMEM((GU + L,), jnp.int32),
            pltpu.VMEM((GU, D), jnp.float32),
            pltpu.VMEM((GU, D), jnp.float32),
            pltpu.VMEM((L,), jnp.int32),
            pltpu.SemaphoreType.DMA,
            pltpu.SemaphoreType.DMA,
        ],
    )(_sc_max_body)


_BR = 1280


def _tc1_body(x_ref, w_ref, deg_ref, o_ref):
    dis = lax.rsqrt(deg_ref[...])
    o_ref[...] = dis * jnp.dot(x_ref[...], w_ref[...],
                               preferred_element_type=jnp.float32)


def _tc2_body(m_ref, deg_ref, b_ref, w_ref, o_ref):
    dis = lax.rsqrt(deg_ref[...])
    h = jnp.maximum(dis * m_ref[...] + b_ref[...], 0.0)
    o_ref[...] = dis * jnp.dot(h, w_ref[...],
                               preferred_element_type=jnp.float32)


def _tc3_body(m_ref, deg_ref, b_ref, w_ref, blin_ref, o_ref, *, n_classes):
    dis = lax.rsqrt(deg_ref[...])
    h = jnp.maximum(dis * m_ref[...] + b_ref[...], 0.0)
    logits = jnp.dot(h, w_ref[...], preferred_element_type=jnp.float32)
    logits = logits + blin_ref[...]
    colid = lax.broadcasted_iota(jnp.int32, logits.shape, 1)
    valid = colid < n_classes
    neg = jnp.float32(-3.0e38)
    logits = jnp.where(valid, logits, neg)
    mx = jnp.max(logits, axis=-1, keepdims=True)
    e = jnp.where(valid, jnp.exp(logits - mx), 0.0)
    s = jnp.sum(e, axis=-1, keepdims=True)
    o_ref[...] = logits - mx - jnp.log(s)


def _tc_call(body, nrows, ins, row_blocked, out_cols=D):
    grid = (nrows // _BR,)
    in_specs = []
    for a, blocked in zip(ins, row_blocked):
        if blocked:
            in_specs.append(pl.BlockSpec((_BR, a.shape[1]), lambda i: (i, 0)))
        else:
            in_specs.append(pl.BlockSpec(a.shape, lambda i: (0, 0)))
    return pl.pallas_call(
        body,
        grid=grid,
        in_specs=in_specs,
        out_specs=pl.BlockSpec((_BR, out_cols), lambda i: (i, 0)),
        out_shape=jax.ShapeDtypeStruct((nrows, out_cols), jnp.float32),
    )(*ins)


def kernel(x, edge_index, W1, b1, W2, b2, Wlin, blin):
    n, d = x.shape
    e = edge_index.shape[1]
    n_classes = Wlin.shape[1]
    assert n == N and d == D

    e_pad = ((e + 4 * CHUNKP - 1) // (4 * CHUNKP)) * (4 * CHUNKP)
    e_half = e_pad // 2
    cap = e_half + 8 * (e_half // CHUNKP) + CHUNKP + PADF
    cap = ((cap + 7) // 8) * 8

    row = edge_index[0].astype(jnp.int32)
    col = edge_index[1].astype(jnp.int32)
    row = jnp.concatenate(
        [row, jnp.zeros((e_pad + CHUNKP - e,), jnp.int32)])
    col = jnp.concatenate(
        [col, jnp.full((e_pad - e,), 1 << 20, jnp.int32),
         jnp.zeros((CHUNKP,), jnp.int32)])

    xp = jnp.pad(x, ((0, NPAD - n), (0, 0)))

    pck, cnts, deg, _ = _make_sc_pre(cap)(col, row)
    degc = deg.reshape(NPAD, 1)

    b1r = b1.reshape(1, D)
    b2r = b2.reshape(1, D)
    wl = jnp.pad(Wlin, ((0, 0), (0, D - n_classes)))
    blr = jnp.pad(blin, (0, D - n_classes)).reshape(1, D)

    sc_max = _make_sc_max(cap)

    g1 = _tc_call(_tc1_body, NPAD, (xp, W1, degc), (True, False, True))
    m1, _ = sc_max(g1, pck, cnts)
    g2 = _tc_call(_tc2_body, NPAD, (m1, degc, b1r, W2),
                  (True, True, False, False))
    m2, _ = sc_max(g2, pck, cnts)
    out = _tc_call(functools.partial(_tc3_body, n_classes=n_classes),
                   NPAD, (m2, degc, b2r, wl, blr),
                   (True, True, False, False, False))
    return out[:n, :n_classes]
